# baseline (speedup 1.0000x reference)
import functools

import jax
import jax.numpy as jnp
import numpy as np
from jax import lax
from jax.experimental import pallas as pl
from jax.experimental.pallas import tpu as pltpu
from jax.experimental.pallas import tpu_sc as plsc


def _lane_bcast(vec, lane):
    idx = jnp.full((16, 1), lane, jnp.int32)
    dnums = lax.GatherDimensionNumbers(
        offset_dims=(), collapsed_slice_dims=(0,), start_index_map=(0,))
    return lax.gather(vec, idx, dnums, (1,),
                      mode=lax.GatherScatterMode.PROMISE_IN_BOUNDS)


_NC = 2
_NS = 16
_CHUNK = 128
_IDXBITS = 14


def _make_spmm(n_nodes, d_half, n_chunks, d_out_pad=None):
    unroll = 4 if d_half <= 32 else 2
    assert n_chunks % unroll == 0
    mesh = plsc.VectorSubcoreMesh(core_axis_name="c", subcore_axis_name="s",
                                  num_cores=_NC, num_subcores=_NS)
    rows_per_tile = (n_nodes // _NS) // 8 * 8
    tail_base = rows_per_tile * _NS
    tail_rows = n_nodes - tail_base
    per_tile = n_chunks * _CHUNK
    d_out_pad = d_out_pad or 2 * d_half

    @functools.partial(
        pl.kernel,
        out_type=jax.ShapeDtypeStruct((n_nodes, d_out_pad), jnp.float32),
        mesh=mesh,
        scratch_types=[
            pltpu.VMEM_SHARED((n_nodes, d_half), jnp.float32),
            pltpu.VMEM((per_tile,), jnp.int32),
            pltpu.VMEM((per_tile,), jnp.float32),
            *[pltpu.VMEM((_CHUNK,), jnp.int32)
              for _ in range(unroll)],
            pltpu.VMEM((_CHUNK,), jnp.int32),
            pltpu.VMEM((_CHUNK,), jnp.int32),
            *[pltpu.VMEM((_CHUNK, d_half), jnp.bfloat16)
              for _ in range(unroll)],
            pltpu.VMEM((_CHUNK, d_half), jnp.float32),
            pltpu.VMEM((_CHUNK, d_half), jnp.float32),
            *[pltpu.SemaphoreType.DMA
              for _ in range(unroll)],
            pltpu.SemaphoreType.DMA,
            pltpu.SemaphoreType.DMA,
        ],
        compiler_params=pltpu.CompilerParams(use_tc_tiling_on_sc=False,
                                             needs_layout_passes=False),
    )
    def spmm(h_hbm, ep_hbm, w_hbm, zeros_hbm, out_hbm,
             acc_sh, ep_v, w_v, *rest):
        sidx = rest[:unroll]
        didx = rest[unroll:unroll + 2]
        gbufs = rest[unroll + 2:2 * unroll + 2]
        sbufs = rest[2 * unroll + 2:2 * unroll + 4]
        gsems = rest[2 * unroll + 4:3 * unroll + 4]
        ssems = rest[3 * unroll + 4:3 * unroll + 6]
        c = lax.axis_index("c")
        s = lax.axis_index("s")
        row_ofs = c * n_nodes

        pltpu.sync_copy(zeros_hbm.at[pl.ds(s * rows_per_tile, rows_per_tile)],
                        acc_sh.at[pl.ds(s * rows_per_tile, rows_per_tile)])
        if tail_rows:
            @pl.when(s == _NS - 1)
            def _():
                pltpu.sync_copy(zeros_hbm.at[pl.ds(tail_base, tail_rows)],
                                acc_sh.at[pl.ds(tail_base, tail_rows)])

        pltpu.sync_copy(ep_hbm.at[pl.ds(s * per_tile, per_tile)], ep_v)
        pltpu.sync_copy(w_hbm.at[pl.ds(s * per_tile, per_tile)], w_v)
        plsc.subcore_barrier()

        def build_sidx(chunk, buf):
            for g16 in range(_CHUNK // 16):
                sl = pl.ds(g16 * 16, 16)
                v = ep_v[pl.ds(chunk * _CHUNK + g16 * 16, 16)]
                buf[sl] = (v & ((1 << _IDXBITS) - 1)) + row_ofs

        def build_didx(chunk, buf):
            for g16 in range(_CHUNK // 16):
                sl = pl.ds(g16 * 16, 16)
                v = ep_v[pl.ds(chunk * _CHUNK + g16 * 16, 16)]
                buf[sl] = lax.shift_right_logical(v, _IDXBITS)

        for b in range(unroll):
            build_sidx(jnp.int32(b), sidx[b])
            pltpu.async_copy(h_hbm.at[sidx[b]], gbufs[b], gsems[b])

        def group_body(jg, carry):
            for b in range(unroll):
                chunk = jg * unroll + b
                b2 = b % 2
                gb, sb = gbufs[b], sbufs[b2]
                pltpu.make_async_copy(h_hbm.at[sidx[b]], gb, gsems[b]).wait()

                @pl.when(chunk >= 2)
                def _():
                    pltpu.make_async_copy(sb, acc_sh.at[didx[b2]],
                                          ssems[b2]).wait()

                for g in range(_CHUNK // 16):
                    wv = w_v[pl.ds(chunk * _CHUNK + g * 16, 16)]
                    for e16 in range(16):
                        e = g * 16 + e16
                        wb = _lane_bcast(wv, e16)
                        for f in range(d_half // 32):
                            v = plsc.bitcast(gb[e, pl.ds(f * 32, 32)],
                                             jnp.int32)
                            lo = plsc.bitcast(lax.shift_left(v, 16),
                                              jnp.float32)
                            hi = plsc.bitcast(v & jnp.int32(-65536),
                                              jnp.float32)
                            sb[e, pl.ds(f * 32, 16)] = lo * wb
                            sb[e, pl.ds(f * 32 + 16, 16)] = hi * wb

                @pl.when(chunk + unroll < n_chunks)
                def _():
                    build_sidx(chunk + unroll, sidx[b])
                    pltpu.async_copy(h_hbm.at[sidx[b]], gb, gsems[b])

                build_didx(chunk, didx[b2])
                pltpu.async_copy(sb, acc_sh.at[didx[b2]], ssems[b2], add=True)
            return carry

        lax.fori_loop(0, n_chunks // unroll, group_body, 0)
        for b in range(2):
            pltpu.make_async_copy(sbufs[b], acc_sh.at[didx[b]],
                                  ssems[b]).wait()

        plsc.subcore_barrier()
        col = pl.ds(c * d_half, d_half)
        pltpu.sync_copy(acc_sh.at[pl.ds(s * rows_per_tile, rows_per_tile)],
                        out_hbm.at[pl.ds(s * rows_per_tile, rows_per_tile),
                                   col])
        if tail_rows:
            @pl.when(s == _NS - 1)
            def _():
                pltpu.sync_copy(acc_sh.at[pl.ds(tail_base, tail_rows)],
                                out_hbm.at[pl.ds(tail_base, tail_rows), col])

    return spmm


def _pack_colorder(dh):
    co = np.empty(dh, np.int64)
    for f in range(dh // 32):
        for l in range(16):
            co[32 * f + 2 * l] = 32 * f + l
            co[32 * f + 2 * l + 1] = 32 * f + 16 + l
    return co


def _nbr_matmul(x, W_nbr, block_n=2000, tanh_in=False):
    xs = x if isinstance(x, tuple) else (x,)
    n, d_in = xs[0].shape
    d_out = W_nbr.shape[1]
    dh = d_out // 2

    co = _pack_colorder(dh)
    W2 = jnp.stack([W_nbr[:, :dh][:, co], W_nbr[:, dh:][:, co]])

    def body(*refs):
        (*x_refs, wn_ref, h_ref) = refs
        if tanh_in:
            xb = jnp.tanh(x_refs[0][...] + x_refs[1][...])
        else:
            xb = x_refs[0][...]
        h_ref[...] = jnp.dot(
            xb, wn_ref[0], preferred_element_type=jnp.float32
        ).astype(jnp.bfloat16)

    return pl.pallas_call(
        body,
        grid=(2, n // block_n),
        in_specs=[
            *[pl.BlockSpec((block_n, d_in), lambda j, i: (i, 0)) for _ in xs],
            pl.BlockSpec((1, d_in, dh), lambda j, i: (j, 0, 0)),
        ],
        out_specs=pl.BlockSpec((block_n, dh),
                               lambda j, i: (j * (n // block_n) + i, 0)),
        out_shape=jax.ShapeDtypeStruct((2 * n, dh), jnp.bfloat16),
    )(*xs, W2)


def _edge_pack(edge_index, edge_weight, e_pad):
    _, e = edge_index.shape
    rows, rows_pad = e // 128, e_pad // 128

    def body(idx_ref, w_ref, o_ref, wo_ref):
        o_ref[pl.ds(0, rows), :] = idx_ref[1] | (idx_ref[0] << _IDXBITS)
        wo_ref[pl.ds(0, rows), :] = w_ref[...]
        if rows_pad > rows:
            zrows = rows_pad - rows
            o_ref[pl.ds(rows, zrows), :] = jnp.zeros((zrows, 128), jnp.int32)
            wo_ref[pl.ds(rows, zrows), :] = jnp.zeros((zrows, 128),
                                                      jnp.float32)

    packed, wp = pl.pallas_call(
        body,
        in_specs=[
            pl.BlockSpec((2, rows, 128), lambda: (0, 0, 0)),
            pl.BlockSpec((rows, 128), lambda: (0, 0)),
        ],
        out_specs=[
            pl.BlockSpec((rows_pad, 128), lambda: (0, 0)),
            pl.BlockSpec((rows_pad, 128), lambda: (0, 0)),
        ],
        out_shape=[
            jax.ShapeDtypeStruct((rows_pad, 128), jnp.int32),
            jax.ShapeDtypeStruct((rows_pad, 128), jnp.float32),
        ],
    )(edge_index.reshape(2, rows, 128),
      edge_weight.astype(jnp.float32).reshape(rows, 128))
    return packed.reshape(e_pad), wp.reshape(e_pad)


def _own_matmul(x, W_own, b, block_n=2000, tanh_in=False):
    xs = x if isinstance(x, tuple) else (x,)
    n, d_in = xs[0].shape
    d_out = W_own.shape[1]

    def body(*refs):
        (*x_refs, wo_ref, b_ref, o_ref) = refs
        if tanh_in:
            xb = jnp.tanh(x_refs[0][...] + x_refs[1][...])
        else:
            xb = x_refs[0][...]
        o_ref[...] = (
            jnp.dot(xb, wo_ref[...], preferred_element_type=jnp.float32)
            + b_ref[...]
        )

    return pl.pallas_call(
        body,
        grid=(n // block_n,),
        in_specs=[
            *[pl.BlockSpec((block_n, d_in), lambda i: (i, 0)) for _ in xs],
            pl.BlockSpec((d_in, d_out), lambda i: (0, 0)),
            pl.BlockSpec((1, d_out), lambda i: (0, 0)),
        ],
        out_specs=pl.BlockSpec((block_n, d_out), lambda i: (i, 0)),
        out_shape=jax.ShapeDtypeStruct((n, d_out), jnp.float32),
    )(*xs, W_own, b.reshape(1, d_out))


def _final_add(agg, own, block_n=1000):
    n, d = own.shape
    dp = agg.shape[1]

    def body(a_ref, own_ref, o_ref):
        o_ref[...] = a_ref[:, :d] + own_ref[...]

    return pl.pallas_call(
        body,
        grid=(n // block_n,),
        in_specs=[
            pl.BlockSpec((block_n, dp), lambda i: (i, 0)),
            pl.BlockSpec((block_n, d), lambda i: (i, 0)),
        ],
        out_specs=pl.BlockSpec((block_n, d), lambda i: (i, 0)),
        out_shape=jax.ShapeDtypeStruct((n, d), jnp.float32),
    )(agg, own)


def kernel(x, edge_index, edge_weight, W_own0, W_nbr0, b0, W_own1, W_nbr1, b1):
    n, d_in = x.shape
    e = edge_weight.shape[0]
    d_hid = W_nbr0.shape[1]
    d_out = W_nbr1.shape[1]

    n_chunks = -(-e // (_NS * _CHUNK))
    n_chunks = -(-n_chunks // 4) * 4
    per_tile = n_chunks * _CHUNK
    e_pad = per_tile * _NS

    ep, w = _edge_pack(edge_index.astype(jnp.int32), edge_weight, e_pad)

    zeros_hid = jnp.zeros((n, d_hid // 2), jnp.float32)
    zeros_out = jnp.zeros((n, d_out // 2), jnp.float32)

    spmm0 = _make_spmm(n, d_hid // 2, n_chunks)
    spmm1 = _make_spmm(n, d_out // 2, n_chunks, d_out_pad=128)

    h0 = _nbr_matmul(x, W_nbr0)
    agg0 = spmm0(h0, ep, w, zeros_hid)
    own0 = _own_matmul(x, W_own0, b0)
    h1 = _nbr_matmul((agg0, own0), W_nbr1, tanh_in=True)
    agg1 = spmm1(h1, ep, w, zeros_out)
    own1 = _own_matmul((agg0, own0), W_own1, b1, tanh_in=True)
    return _final_add(agg1, own1)


# speedup vs baseline: 1.1234x; 1.1234x over previous
import functools

import jax
import jax.numpy as jnp
import numpy as np
from jax import lax
from jax.experimental import pallas as pl
from jax.experimental.pallas import tpu as pltpu
from jax.experimental.pallas import tpu_sc as plsc


def _lane_bcast(vec, lane):
    idx = jnp.full((16, 1), lane, jnp.int32)
    dnums = lax.GatherDimensionNumbers(
        offset_dims=(), collapsed_slice_dims=(0,), start_index_map=(0,))
    return lax.gather(vec, idx, dnums, (1,),
                      mode=lax.GatherScatterMode.PROMISE_IN_BOUNDS)


_NC = 2
_NS = 16
_CHUNK = 128
_IDXBITS = 14


def _make_spmm(n_nodes, d_half, n_chunks, d_out_pad=None):
    unroll = 2
    assert n_chunks % unroll == 0
    mesh = plsc.VectorSubcoreMesh(core_axis_name="c", subcore_axis_name="s",
                                  num_cores=_NC, num_subcores=_NS)
    rows_per_tile = (n_nodes // _NS) // 8 * 8
    tail_base = rows_per_tile * _NS
    tail_rows = n_nodes - tail_base
    per_tile = n_chunks * _CHUNK
    d_out_pad = d_out_pad or 2 * d_half

    @functools.partial(
        pl.kernel,
        out_type=jax.ShapeDtypeStruct((n_nodes, d_out_pad), jnp.float32),
        mesh=mesh,
        scratch_types=[
            pltpu.VMEM_SHARED((n_nodes, d_half), jnp.float32),
            pltpu.VMEM((per_tile,), jnp.int32),
            pltpu.VMEM((per_tile,), jnp.float32),
            *[pltpu.VMEM((_CHUNK,), jnp.int32)
              for _ in range(unroll)],
            pltpu.VMEM((_CHUNK,), jnp.int32),
            pltpu.VMEM((_CHUNK,), jnp.int32),
            *[pltpu.VMEM((_CHUNK, d_half), jnp.bfloat16)
              for _ in range(unroll)],
            pltpu.VMEM((_CHUNK, d_half), jnp.float32),
            pltpu.VMEM((_CHUNK, d_half), jnp.float32),
            *[pltpu.SemaphoreType.DMA
              for _ in range(unroll)],
            pltpu.SemaphoreType.DMA,
            pltpu.SemaphoreType.DMA,
        ],
        compiler_params=pltpu.CompilerParams(use_tc_tiling_on_sc=False,
                                             needs_layout_passes=False),
    )
    def spmm(h_hbm, ep_hbm, w_hbm, zeros_hbm, out_hbm,
             acc_sh, ep_v, w_v, *rest):
        sidx = rest[:unroll]
        didx = rest[unroll:unroll + 2]
        gbufs = rest[unroll + 2:2 * unroll + 2]
        sbufs = rest[2 * unroll + 2:2 * unroll + 4]
        gsems = rest[2 * unroll + 4:3 * unroll + 4]
        ssems = rest[3 * unroll + 4:3 * unroll + 6]
        c = lax.axis_index("c")
        s = lax.axis_index("s")
        row_ofs = c * n_nodes

        pltpu.sync_copy(zeros_hbm.at[pl.ds(s * rows_per_tile, rows_per_tile)],
                        acc_sh.at[pl.ds(s * rows_per_tile, rows_per_tile)])
        if tail_rows:
            @pl.when(s == _NS - 1)
            def _():
                pltpu.sync_copy(zeros_hbm.at[pl.ds(tail_base, tail_rows)],
                                acc_sh.at[pl.ds(tail_base, tail_rows)])

        pltpu.sync_copy(ep_hbm.at[pl.ds(s * per_tile, per_tile)], ep_v)
        pltpu.sync_copy(w_hbm.at[pl.ds(s * per_tile, per_tile)], w_v)
        plsc.subcore_barrier()

        def build_sidx(chunk, buf):
            for g16 in range(_CHUNK // 16):
                sl = pl.ds(g16 * 16, 16)
                v = ep_v[pl.ds(chunk * _CHUNK + g16 * 16, 16)]
                buf[sl] = (v & ((1 << _IDXBITS) - 1)) + row_ofs

        def build_didx(chunk, buf):
            for g16 in range(_CHUNK // 16):
                sl = pl.ds(g16 * 16, 16)
                v = ep_v[pl.ds(chunk * _CHUNK + g16 * 16, 16)]
                buf[sl] = lax.shift_right_logical(v, _IDXBITS)

        for b in range(unroll):
            build_sidx(jnp.int32(b), sidx[b])
            pltpu.async_copy(h_hbm.at[sidx[b]], gbufs[b], gsems[b])

        def group_body(jg, carry):
            for b in range(unroll):
                chunk = jg * unroll + b
                b2 = b % 2
                gb, sb = gbufs[b], sbufs[b2]
                pltpu.make_async_copy(h_hbm.at[sidx[b]], gb, gsems[b]).wait()

                @pl.when(chunk >= 2)
                def _():
                    pltpu.make_async_copy(sb, acc_sh.at[didx[b2]],
                                          ssems[b2]).wait()

                for g in range(_CHUNK // 16):
                    wv = w_v[pl.ds(chunk * _CHUNK + g * 16, 16)]
                    for e16 in range(16):
                        e = g * 16 + e16
                        wb = _lane_bcast(wv, e16)
                        for f in range(d_half // 32):
                            v = plsc.bitcast(gb[e, pl.ds(f * 32, 32)],
                                             jnp.int32)
                            lo = plsc.bitcast(lax.shift_left(v, 16),
                                              jnp.float32)
                            hi = plsc.bitcast(v & jnp.int32(-65536),
                                              jnp.float32)
                            sb[e, pl.ds(f * 32, 16)] = lo * wb
                            sb[e, pl.ds(f * 32 + 16, 16)] = hi * wb

                @pl.when(chunk + unroll < n_chunks)
                def _():
                    build_sidx(chunk + unroll, sidx[b])
                    pltpu.async_copy(h_hbm.at[sidx[b]], gb, gsems[b])

                build_didx(chunk, didx[b2])
                pltpu.async_copy(sb, acc_sh.at[didx[b2]], ssems[b2], add=True)
            return carry

        lax.fori_loop(0, n_chunks // unroll, group_body, 0)
        for b in range(2):
            pltpu.make_async_copy(sbufs[b], acc_sh.at[didx[b]],
                                  ssems[b]).wait()

        plsc.subcore_barrier()
        col = pl.ds(c * d_half, d_half)
        pltpu.sync_copy(acc_sh.at[pl.ds(s * rows_per_tile, rows_per_tile)],
                        out_hbm.at[pl.ds(s * rows_per_tile, rows_per_tile),
                                   col])
        if tail_rows:
            @pl.when(s == _NS - 1)
            def _():
                pltpu.sync_copy(acc_sh.at[pl.ds(tail_base, tail_rows)],
                                out_hbm.at[pl.ds(tail_base, tail_rows), col])

    return spmm


def _pack_colorder(dh):
    co = np.empty(dh, np.int64)
    for f in range(dh // 32):
        for l in range(16):
            co[32 * f + 2 * l] = 32 * f + l
            co[32 * f + 2 * l + 1] = 32 * f + 16 + l
    return co


def _nbr_matmul(x, W_nbr, block_n=1000, tanh_in=False):
    xs = x if isinstance(x, tuple) else (x,)
    n, d_in = xs[0].shape
    d_out = W_nbr.shape[1]
    dh = d_out // 2

    co = _pack_colorder(dh)
    W2 = jnp.stack([W_nbr[:, :dh][:, co], W_nbr[:, dh:][:, co]])

    def body(*refs):
        (*x_refs, wn_ref, h_ref) = refs
        if tanh_in:
            xb = jnp.tanh(x_refs[0][...] + x_refs[1][...])
        else:
            xb = x_refs[0][...]
        h_ref[...] = jnp.dot(
            xb, wn_ref[0], preferred_element_type=jnp.float32
        ).astype(jnp.bfloat16)

    return pl.pallas_call(
        body,
        grid=(2, n // block_n),
        in_specs=[
            *[pl.BlockSpec((block_n, d_in), lambda j, i: (i, 0)) for _ in xs],
            pl.BlockSpec((1, d_in, dh), lambda j, i: (j, 0, 0)),
        ],
        out_specs=pl.BlockSpec((block_n, dh),
                               lambda j, i: (j * (n // block_n) + i, 0)),
        out_shape=jax.ShapeDtypeStruct((2 * n, dh), jnp.bfloat16),
    )(*xs, W2)


def _edge_pack(edge_index, edge_weight, e_pad):
    _, e = edge_index.shape
    rows, rows_pad = e // 128, e_pad // 128

    def body(idx_ref, w_ref, o_ref, wo_ref):
        o_ref[pl.ds(0, rows), :] = idx_ref[1] | (idx_ref[0] << _IDXBITS)
        wo_ref[pl.ds(0, rows), :] = w_ref[...]
        if rows_pad > rows:
            zrows = rows_pad - rows
            o_ref[pl.ds(rows, zrows), :] = jnp.zeros((zrows, 128), jnp.int32)
            wo_ref[pl.ds(rows, zrows), :] = jnp.zeros((zrows, 128),
                                                      jnp.float32)

    packed, wp = pl.pallas_call(
        body,
        in_specs=[
            pl.BlockSpec((2, rows, 128), lambda: (0, 0, 0)),
            pl.BlockSpec((rows, 128), lambda: (0, 0)),
        ],
        out_specs=[
            pl.BlockSpec((rows_pad, 128), lambda: (0, 0)),
            pl.BlockSpec((rows_pad, 128), lambda: (0, 0)),
        ],
        out_shape=[
            jax.ShapeDtypeStruct((rows_pad, 128), jnp.int32),
            jax.ShapeDtypeStruct((rows_pad, 128), jnp.float32),
        ],
    )(edge_index.reshape(2, rows, 128),
      edge_weight.astype(jnp.float32).reshape(rows, 128))
    return packed.reshape(e_pad), wp.reshape(e_pad)


def _own_matmul(x, W_own, b, block_n=1000, tanh_in=False):
    xs = x if isinstance(x, tuple) else (x,)
    n, d_in = xs[0].shape
    d_out = W_own.shape[1]

    def body(*refs):
        (*x_refs, wo_ref, b_ref, o_ref) = refs
        if tanh_in:
            xb = jnp.tanh(x_refs[0][...] + x_refs[1][...])
        else:
            xb = x_refs[0][...]
        o_ref[...] = (
            jnp.dot(xb, wo_ref[...], preferred_element_type=jnp.float32)
            + b_ref[...]
        )

    return pl.pallas_call(
        body,
        grid=(n // block_n,),
        in_specs=[
            *[pl.BlockSpec((block_n, d_in), lambda i: (i, 0)) for _ in xs],
            pl.BlockSpec((d_in, d_out), lambda i: (0, 0)),
            pl.BlockSpec((1, d_out), lambda i: (0, 0)),
        ],
        out_specs=pl.BlockSpec((block_n, d_out), lambda i: (i, 0)),
        out_shape=jax.ShapeDtypeStruct((n, d_out), jnp.float32),
    )(*xs, W_own, b.reshape(1, d_out))


def _final_add(agg, own, block_n=1000):
    n, d = own.shape
    dp = agg.shape[1]

    def body(a_ref, own_ref, o_ref):
        o_ref[...] = a_ref[:, :d] + own_ref[...]

    return pl.pallas_call(
        body,
        grid=(n // block_n,),
        in_specs=[
            pl.BlockSpec((block_n, dp), lambda i: (i, 0)),
            pl.BlockSpec((block_n, d), lambda i: (i, 0)),
        ],
        out_specs=pl.BlockSpec((block_n, d), lambda i: (i, 0)),
        out_shape=jax.ShapeDtypeStruct((n, d), jnp.float32),
    )(agg, own)


def kernel(x, edge_index, edge_weight, W_own0, W_nbr0, b0, W_own1, W_nbr1, b1):
    n, d_in = x.shape
    e = edge_weight.shape[0]
    d_hid = W_nbr0.shape[1]
    d_out = W_nbr1.shape[1]

    n_chunks = -(-e // (_NS * _CHUNK))
    n_chunks += n_chunks % 2
    per_tile = n_chunks * _CHUNK
    e_pad = per_tile * _NS

    ep, w = _edge_pack(edge_index.astype(jnp.int32), edge_weight, e_pad)

    zeros_hid = jnp.zeros((n, d_hid // 2), jnp.float32)
    zeros_out = jnp.zeros((n, d_out // 2), jnp.float32)

    spmm0 = _make_spmm(n, d_hid // 2, n_chunks)
    spmm1 = _make_spmm(n, d_out // 2, n_chunks, d_out_pad=128)

    h0 = _nbr_matmul(x, W_nbr0)
    agg0 = spmm0(h0, ep, w, zeros_hid)
    own0 = _own_matmul(x, W_own0, b0)
    h1 = _nbr_matmul((agg0, own0), W_nbr1, tanh_in=True)
    agg1 = spmm1(h1, ep, w, zeros_out)
    own1 = _own_matmul((agg0, own0), W_own1, b1, tanh_in=True)
    return _final_add(agg1, own1)
